# baseline (speedup 1.0000x reference)
import functools

import jax
import jax.numpy as jnp
from jax import lax
from jax.experimental import pallas as pl
from jax.experimental.pallas import tpu as pltpu
from jax.experimental.pallas import tpu_sc as plsc

HIDDEN = 1024
SLOTS = 65536
N = 64
TOPK = 4
ETA = 0.1
EPS = 1e-8

BS = 2048
NB = SLOTS // BS
H2 = BS // 2
KPAD = 8
NEG = -1e30


def _topk_body(x_ref, wa_ref, alo_ref, ahi_ref, outv_ref, outi_ref,
               qn_ref, rv_ref, ri_ref):
    j = pl.program_id(0)

    @pl.when(j == 0)
    def _init():
        q = lax.dot_general(x_ref[...], wa_ref[...], (((1,), (1,)), ((), ())),
                            preferred_element_type=jnp.float32)
        qnorm = jnp.sqrt(jnp.sum(q * q, axis=1, keepdims=True))
        qn_ref[...] = q / jnp.clip(qnorm, EPS, None)
        rv_ref[...] = jnp.full((KPAD, N), NEG, jnp.float32)
        ri_ref[...] = jnp.zeros((KPAD, N), jnp.int32)

    qn = qn_ref[...]
    a_lo = alo_ref[...]
    nrm_lo = jnp.sqrt(jnp.sum(a_lo * a_lo, axis=1, keepdims=True))
    an_lo = a_lo / jnp.clip(nrm_lo, EPS, None)
    s_lo = lax.dot_general(an_lo, qn, (((1,), (1,)), ((), ())),
                           preferred_element_type=jnp.float32)
    a_hi = ahi_ref[...]
    nrm_hi = jnp.sqrt(jnp.sum(a_hi * a_hi, axis=1, keepdims=True))
    an_hi = a_hi / jnp.clip(nrm_hi, EPS, None)
    s_hi = lax.dot_general(an_hi, qn, (((1,), (1,)), ((), ())),
                           preferred_element_type=jnp.float32)
    s2 = jnp.concatenate([s_lo, s_hi], axis=1)

    iota_s = lax.broadcasted_iota(jnp.int32, (H2, 2 * N), 0)
    bv = []
    bi = []
    for t in range(TOPK):
        m = jnp.max(s2, axis=0)
        hit = s2 == m[None, :]
        am = jnp.min(jnp.where(hit, iota_s, SLOTS), axis=0)
        bv.append(m)
        bi.append(am)
        if t < TOPK - 1:
            s2 = jnp.where(iota_s == am[None, :], NEG, s2)

    bvc = jnp.concatenate([v[None, :] for v in bv], axis=0)
    bic = jnp.concatenate([i[None, :] for i in bi], axis=0)
    cat_v = jnp.concatenate(
        [rv_ref[0:TOPK, :], bvc[:, 0:N], bvc[:, N:2 * N]], axis=0)
    cat_i = jnp.concatenate(
        [ri_ref[0:TOPK, :],
         bic[:, 0:N] + j * BS,
         bic[:, N:2 * N] + (j * BS + H2)], axis=0)
    NC12 = 3 * TOPK
    iota_c = lax.broadcasted_iota(jnp.int32, (NC12, N), 0)
    nv = []
    ni = []
    for t in range(TOPK):
        m = jnp.max(cat_v, axis=0)
        hit = cat_v == m[None, :]
        pos = jnp.min(jnp.where(hit, iota_c, NC12), axis=0)
        sel = iota_c == pos[None, :]
        nv.append(m)
        ni.append(jnp.sum(jnp.where(sel, cat_i, 0), axis=0))
        if t < TOPK - 1:
            cat_v = jnp.where(sel, NEG, cat_v)
    pad_v = [jnp.full((N,), NEG, jnp.float32)[None, :]] * (KPAD - TOPK)
    pad_i = [jnp.zeros((N,), jnp.int32)[None, :]] * (KPAD - TOPK)
    rv_ref[...] = jnp.concatenate([v[None, :] for v in nv] + pad_v, axis=0)
    ri_ref[...] = jnp.concatenate([i[None, :] for i in ni] + pad_i, axis=0)

    @pl.when(j == NB - 1)
    def _out():
        outv_ref[...] = rv_ref[...]
        outi_ref[...] = ri_ref[...]


def _topk_call(x, addresses, Wa):
    return pl.pallas_call(
        _topk_body,
        grid=(NB,),
        in_specs=[
            pl.BlockSpec((N, HIDDEN), lambda j: (0, 0)),
            pl.BlockSpec((HIDDEN, HIDDEN), lambda j: (0, 0)),
            pl.BlockSpec((H2, HIDDEN), lambda j: (2 * j, 0)),
            pl.BlockSpec((H2, HIDDEN), lambda j: (2 * j + 1, 0)),
        ],
        out_specs=[
            pl.BlockSpec((KPAD, N), lambda j: (0, 0)),
            pl.BlockSpec((KPAD, N), lambda j: (0, 0)),
        ],
        out_shape=[
            jax.ShapeDtypeStruct((KPAD, N), jnp.float32),
            jax.ShapeDtypeStruct((KPAD, N), jnp.int32),
        ],
        scratch_shapes=[
            pltpu.VMEM((N, HIDDEN), jnp.float32),
            pltpu.VMEM((KPAD, N), jnp.float32),
            pltpu.VMEM((KPAD, N), jnp.int32),
        ],
    )(x, Wa, addresses, addresses)


_NW = 32
_BPW = (N * TOPK) // _NW


def _gather_sc(contents, idx):
    mesh = plsc.VectorSubcoreMesh(core_axis_name="c", subcore_axis_name="s")

    @functools.partial(
        pl.kernel, mesh=mesh,
        out_type=jax.ShapeDtypeStruct((N * TOPK, HIDDEN), jnp.float32),
        scratch_types=[
            pltpu.VMEM((_BPW,), jnp.int32),
            pltpu.VMEM((_BPW, HIDDEN), jnp.float32),
            pltpu.SemaphoreType.DMA,
        ],
    )
    def gk(table_hbm, idx_hbm, out_hbm, idx_v, rows_v, sem):
        wid = lax.axis_index("s") * 2 + lax.axis_index("c")
        base = wid * _BPW
        pltpu.sync_copy(idx_hbm.at[pl.ds(base, _BPW)], idx_v)
        pltpu.async_copy(table_hbm.at[idx_v], rows_v, sem).wait()
        pltpu.sync_copy(rows_v, out_hbm.at[pl.ds(base, _BPW)])

    return gk(contents, idx)


def _combine_body(x_ref, ti_ref, tiT_ref, tvT_ref, g_ref, wr_ref, out_ref):
    x = x_ref[...]
    best_row = ti_ref[0:1, :]
    tvT = tvT_ref[...]
    m = jnp.max(tvT, axis=1, keepdims=True)
    e = jnp.exp(tvT - m)
    w = e / jnp.sum(e, axis=1, keepdims=True)

    read = jnp.zeros((N, HIDDEN), jnp.float32)
    for k in range(TOPK):
        ti_col = tiT_ref[:, k:k + 1]
        match = (ti_col == best_row).astype(jnp.float32)
        counts = jnp.sum(match, axis=1, keepdims=True)
        sums = lax.dot_general(match, x, (((1,), (0,)), ((), ())),
                               preferred_element_type=jnp.float32)
        mask = (counts > 0).astype(jnp.float32)
        mean_w = sums / jnp.clip(counts, 1.0, None)
        g_k = g_ref[k * N:(k + 1) * N, :]
        upd = g_k * (1.0 - ETA * mask) + ETA * mask * mean_w
        read = read + w[:, k:k + 1] * upd
    out_ref[...] = lax.dot_general(read, wr_ref[...], (((1,), (1,)), ((), ())),
                                   preferred_element_type=jnp.float32)


def _combine_call(x, topi, topiT, topvT, gathered, Wr):
    return pl.pallas_call(
        _combine_body,
        out_shape=jax.ShapeDtypeStruct((N, HIDDEN), jnp.float32),
    )(x, topi, topiT, topvT, gathered, Wr)


def kernel(x, addresses, contents, Wa, Wr):
    topv, topi = _topk_call(x, addresses, Wa)
    idx_flat = topi[0:TOPK, :].reshape(-1)
    gathered = _gather_sc(contents, idx_flat)
    topiT = topi[0:TOPK, :].T
    topvT = topv[0:TOPK, :].T
    return _combine_call(x, topi, topiT, topvT, gathered, Wr)


# speedup vs baseline: 1.0750x; 1.0750x over previous
import functools

import jax
import jax.numpy as jnp
from jax import lax
from jax.experimental import pallas as pl
from jax.experimental.pallas import tpu as pltpu
from jax.experimental.pallas import tpu_sc as plsc

HIDDEN = 1024
SLOTS = 65536
N = 64
TOPK = 4
ETA = 0.1
EPS = 1e-8

BS = 4096
NB = SLOTS // BS
H2 = BS // 2
KPAD = 8
NEG = -1e30


def _topk_body(x_ref, wa_ref, alo_ref, ahi_ref, outv_ref, outi_ref,
               qn_ref, rv_ref, ri_ref):
    j = pl.program_id(0)

    @pl.when(j == 0)
    def _init():
        q = lax.dot_general(x_ref[...], wa_ref[...], (((1,), (1,)), ((), ())),
                            preferred_element_type=jnp.float32)
        qnorm = jnp.sqrt(jnp.sum(q * q, axis=1, keepdims=True))
        qn_ref[...] = q / jnp.clip(qnorm, EPS, None)
        rv_ref[...] = jnp.full((KPAD, N), NEG, jnp.float32)
        ri_ref[...] = jnp.zeros((KPAD, N), jnp.int32)

    qn = qn_ref[...]
    a_lo = alo_ref[...]
    nrm_lo = jnp.sqrt(jnp.sum(a_lo * a_lo, axis=1, keepdims=True))
    an_lo = a_lo / jnp.clip(nrm_lo, EPS, None)
    s_lo = lax.dot_general(an_lo, qn, (((1,), (1,)), ((), ())),
                           preferred_element_type=jnp.float32)
    a_hi = ahi_ref[...]
    nrm_hi = jnp.sqrt(jnp.sum(a_hi * a_hi, axis=1, keepdims=True))
    an_hi = a_hi / jnp.clip(nrm_hi, EPS, None)
    s_hi = lax.dot_general(an_hi, qn, (((1,), (1,)), ((), ())),
                           preferred_element_type=jnp.float32)
    s2 = jnp.concatenate([s_lo, s_hi], axis=1)

    iota_s = lax.broadcasted_iota(jnp.int32, (H2, 2 * N), 0)
    bv = []
    bi = []
    for t in range(TOPK):
        m = jnp.max(s2, axis=0)
        hit = s2 == m[None, :]
        am = jnp.min(jnp.where(hit, iota_s, SLOTS), axis=0)
        bv.append(m)
        bi.append(am)
        if t < TOPK - 1:
            s2 = jnp.where(iota_s == am[None, :], NEG, s2)

    bvc = jnp.concatenate([v[None, :] for v in bv], axis=0)
    bic = jnp.concatenate([i[None, :] for i in bi], axis=0)
    cat_v = jnp.concatenate(
        [rv_ref[0:TOPK, :], bvc[:, 0:N], bvc[:, N:2 * N]], axis=0)
    cat_i = jnp.concatenate(
        [ri_ref[0:TOPK, :],
         bic[:, 0:N] + j * BS,
         bic[:, N:2 * N] + (j * BS + H2)], axis=0)
    NC12 = 3 * TOPK
    iota_c = lax.broadcasted_iota(jnp.int32, (NC12, N), 0)
    nv = []
    ni = []
    for t in range(TOPK):
        m = jnp.max(cat_v, axis=0)
        hit = cat_v == m[None, :]
        pos = jnp.min(jnp.where(hit, iota_c, NC12), axis=0)
        sel = iota_c == pos[None, :]
        nv.append(m)
        ni.append(jnp.sum(jnp.where(sel, cat_i, 0), axis=0))
        if t < TOPK - 1:
            cat_v = jnp.where(sel, NEG, cat_v)
    pad_v = [jnp.full((N,), NEG, jnp.float32)[None, :]] * (KPAD - TOPK)
    pad_i = [jnp.zeros((N,), jnp.int32)[None, :]] * (KPAD - TOPK)
    rv_ref[...] = jnp.concatenate([v[None, :] for v in nv] + pad_v, axis=0)
    ri_ref[...] = jnp.concatenate([i[None, :] for i in ni] + pad_i, axis=0)

    @pl.when(j == NB - 1)
    def _out():
        outv_ref[...] = rv_ref[...]
        outi_ref[...] = ri_ref[...]


def _topk_call(x, addresses, Wa):
    return pl.pallas_call(
        _topk_body,
        grid=(NB,),
        in_specs=[
            pl.BlockSpec((N, HIDDEN), lambda j: (0, 0)),
            pl.BlockSpec((HIDDEN, HIDDEN), lambda j: (0, 0)),
            pl.BlockSpec((H2, HIDDEN), lambda j: (2 * j, 0)),
            pl.BlockSpec((H2, HIDDEN), lambda j: (2 * j + 1, 0)),
        ],
        out_specs=[
            pl.BlockSpec((KPAD, N), lambda j: (0, 0)),
            pl.BlockSpec((KPAD, N), lambda j: (0, 0)),
        ],
        out_shape=[
            jax.ShapeDtypeStruct((KPAD, N), jnp.float32),
            jax.ShapeDtypeStruct((KPAD, N), jnp.int32),
        ],
        scratch_shapes=[
            pltpu.VMEM((N, HIDDEN), jnp.float32),
            pltpu.VMEM((KPAD, N), jnp.float32),
            pltpu.VMEM((KPAD, N), jnp.int32),
        ],
    )(x, Wa, addresses, addresses)


_NW = 32
_BPW = (N * TOPK) // _NW


def _gather_sc(contents, idx):
    mesh = plsc.VectorSubcoreMesh(core_axis_name="c", subcore_axis_name="s")

    @functools.partial(
        pl.kernel, mesh=mesh,
        out_type=jax.ShapeDtypeStruct((N * TOPK, HIDDEN), jnp.float32),
        scratch_types=[
            pltpu.VMEM((_BPW,), jnp.int32),
            pltpu.VMEM((_BPW, HIDDEN), jnp.float32),
            pltpu.SemaphoreType.DMA,
        ],
    )
    def gk(table_hbm, idx_hbm, out_hbm, idx_v, rows_v, sem):
        wid = lax.axis_index("s") * 2 + lax.axis_index("c")
        base = wid * _BPW
        pltpu.sync_copy(idx_hbm.at[pl.ds(base, _BPW)], idx_v)
        pltpu.async_copy(table_hbm.at[idx_v], rows_v, sem).wait()
        pltpu.sync_copy(rows_v, out_hbm.at[pl.ds(base, _BPW)])

    return gk(contents, idx)


def _combine_body(x_ref, ti_ref, tv_ref, g_ref, wr_ref, out_ref):
    x = x_ref[...]
    best_row = ti_ref[0:1, :]
    tiT = ti_ref[0:TOPK, :].T
    tvT = tv_ref[0:TOPK, :].T
    m = jnp.max(tvT, axis=1, keepdims=True)
    e = jnp.exp(tvT - m)
    w = e / jnp.sum(e, axis=1, keepdims=True)

    read = jnp.zeros((N, HIDDEN), jnp.float32)
    for k in range(TOPK):
        ti_col = tiT[:, k:k + 1]
        match = (ti_col == best_row).astype(jnp.float32)
        counts = jnp.sum(match, axis=1, keepdims=True)
        sums = lax.dot_general(match, x, (((1,), (0,)), ((), ())),
                               preferred_element_type=jnp.float32)
        mask = (counts > 0).astype(jnp.float32)
        mean_w = sums / jnp.clip(counts, 1.0, None)
        g_k = g_ref[k * N:(k + 1) * N, :]
        upd = g_k * (1.0 - ETA * mask) + ETA * mask * mean_w
        read = read + w[:, k:k + 1] * upd
    out_ref[...] = lax.dot_general(read, wr_ref[...], (((1,), (1,)), ((), ())),
                                   preferred_element_type=jnp.float32)


def _combine_call(x, topi, topv, gathered, Wr):
    return pl.pallas_call(
        _combine_body,
        out_shape=jax.ShapeDtypeStruct((N, HIDDEN), jnp.float32),
    )(x, topi, topv, gathered, Wr)


def kernel(x, addresses, contents, Wa, Wr):
    topv, topi = _topk_call(x, addresses, Wa)
    idx_flat = topi[0:TOPK, :].reshape(-1)
    gathered = _gather_sc(contents, idx_flat)
    return _combine_call(x, topi, topv, gathered, Wr)
